# baseline (speedup 1.0000x reference)
import functools

import jax
import jax.numpy as jnp
from jax import lax
from jax.experimental import pallas as pl
from jax.experimental.pallas import tpu as pltpu
from jax.experimental.pallas import tpu_sc as plsc

N = 32768
K = 128
ROWS = 512
NC, NS = 2, 16
NW = NC * NS
RPW = ROWS // NW
NCHUNK = N // 16
PENALTY = -1000000.0
EPS = 1e-5


def _cmpex(a, b):
    ak, av = a
    bk, bv = b
    swap = (bk < ak) | ((bk == ak) & (bv < av))
    lk = jnp.where(swap, bk, ak)
    lv = jnp.where(swap, bv, av)
    hk = jnp.where(swap, ak, bk)
    hv = jnp.where(swap, av, bv)
    return (lk, lv), (hk, hv)


def _bitonic_clean(seq):
    if len(seq) == 1:
        k, v = seq[0]
        sk, sv = plsc.sort_key_val(k, v)
        return [(sk, sv)]
    half = len(seq) // 2
    lo, hi = [], []
    for i in range(half):
        l, h = _cmpex(seq[i], seq[i + half])
        lo.append(l)
        hi.append(h)
    return _bitonic_clean(lo) + _bitonic_clean(hi)


def _merge(a, b):
    rb = [(lax.rev(k, (0,)), lax.rev(v, (0,))) for (k, v) in reversed(b)]
    lo, hi = [], []
    for x, y in zip(a, rb):
        l, h = _cmpex(x, y)
        lo.append(l)
        hi.append(h)
    return _bitonic_clean(lo) + _bitonic_clean(hi)


def _topk_body(dist_ref, theta_ref, sd_ref, ix_ref, th_ref,
               row_v, cv, ci, hist, totals, ctot,
               accv, acci, gidx, thb, sem):
    lane = lax.iota(jnp.int32, 16)
    lane_off = lane * 256
    ones = jnp.ones((16,), jnp.int32)
    zeros16 = jnp.zeros((16,), jnp.int32)
    wid = lax.axis_index("c") * NS + lax.axis_index("s")

    def clear_hist(i, _):
        hist[pl.ds(i * 16, 16)] = zeros16
        return 0

    def scan_hist(kp):
        for c in range(16):
            tv = hist[pl.ds(c * 16, 16)]
            for l in range(1, 16):
                tv = tv + hist[pl.ds(l * 256 + c * 16, 16)]
            totals[pl.ds(c * 16, 16)] = tv
            ctot[c] = jnp.sum(tv)

        def outer(c, st):
            cum, cc, cb = st
            t = ctot[c]
            hit = jnp.logical_and(cum < kp, cum + t >= kp)
            cc = jnp.where(hit, c, cc)
            cb = jnp.where(hit, cum, cb)
            return (cum + t, cc, cb)

        _, cc, cb = lax.fori_loop(0, 16, outer, (0, 0, 0))

        tv = totals[pl.ds(cc * 16, 16)]
        cum_v = plsc.cumsum(tv) + cb
        hitmask = cum_v >= kp
        j = plsc.all_reduce_ffs(hitmask)[0]
        tbin = cc * 16 + j
        c_less = cb + jnp.sum(jnp.where(hitmask, 0, tv))
        return tbin, c_less

    def radix_pass(shift, n, kp, acc_off, first):
        @plsc.parallel_loop(0, 256, unroll=8)
        def _(i):
            hist[pl.ds(i * 16, 16)] = zeros16

        if first:
            @plsc.parallel_loop(0, NCHUNK, unroll=16)
            def _(i):
                v = row_v[pl.ds(i * 16, 16)]
                key = lax.bitcast_convert_type(v, jnp.int32)
                byte = (key >> shift) & 255
                plsc.addupdate_scatter(hist, [lane_off + byte], ones)
            nch = NCHUNK
        else:
            def hbody(i, _):
                base = i * 16
                v = cv[pl.ds(base, 16)]
                key = lax.bitcast_convert_type(v, jnp.int32)
                byte = (key >> shift) & 255
                m = (base + lane) < n
                plsc.addupdate_scatter(hist, [lane_off + byte], ones, mask=m)
                return 0

            nch = pl.cdiv(n, 16)
            lax.fori_loop(0, nch, hbody, 0)

        tbin, c_less = scan_hist(kp)
        tsplat = jnp.full((16,), tbin, jnp.int32)

        def cstep(i, offa, offe, v, iv, m_lt, m_eq):
            r_lt = plsc.cumsum(jnp.where(m_lt, 1, 0))
            r_eq = plsc.cumsum(jnp.where(m_eq, 1, 0))
            pos_a = offa + r_lt - 1
            pos_e = offe + r_eq - 1
            plsc.store_scatter(accv, [pos_a], v, mask=m_lt)
            plsc.store_scatter(acci, [pos_a], iv, mask=m_lt)
            if first:
                plsc.store_scatter(cv, [pos_e], v, mask=m_eq)
                plsc.store_scatter(ci, [pos_e], iv, mask=m_eq)
            else:
                plsc.store_scatter(cv, [pos_e], v, mask=m_eq)
                plsc.store_scatter(ci, [pos_e], iv, mask=m_eq)
            offa = offa + plsc.all_reduce_population_count(m_lt)
            offe = offe + plsc.all_reduce_population_count(m_eq)
            return offa, offe

        offa0 = jnp.full((16,), acc_off, jnp.int32)
        if first:
            @plsc.parallel_loop(0, NCHUNK, unroll=8, carry=(offa0, zeros16))
            def offs_out(i, offs):
                offa, offe = offs
                base = i * 16
                v = row_v[pl.ds(base, 16)]
                iv = base + lane
                key = lax.bitcast_convert_type(v, jnp.int32)
                byte = (key >> shift) & 255
                return cstep(i, offa, offe, v, iv, byte < tsplat,
                             byte == tsplat)
            _, offe_v = offs_out
        else:
            def cbody(i, offs):
                offa, offe = offs
                base = i * 16
                gids = base + lane
                v = cv[pl.ds(base, 16)]
                iv = ci[pl.ds(base, 16)]
                mval = gids < n
                key = lax.bitcast_convert_type(v, jnp.int32)
                byte = (key >> shift) & 255
                m_lt = jnp.logical_and(byte < tsplat, mval)
                m_eq = jnp.logical_and(byte == tsplat, mval)
                return cstep(i, offa, offe, v, iv, m_lt, m_eq)

            _, offe_v = lax.fori_loop(0, nch, cbody, (offa0, zeros16))
        return kp - c_less, offe_v[0], acc_off + c_less

    def do_row(r, _):
        row = wid * RPW + r
        pltpu.sync_copy(dist_ref.at[row], row_v)

        kp, n, off = radix_pass(24, N, 128, 0, True)
        kp, n, off = radix_pass(16, n, kp, off, False)
        kp, n, off = radix_pass(8, n, kp, off, False)
        kp, n, off = radix_pass(0, n, kp, off, False)

        offv = jnp.full((16,), off, jnp.int32)
        for c in range(8):
            base = c * 16
            m = (base + lane) < kp
            pos = offv + base + lane
            plsc.store_scatter(accv, [pos], cv[pl.ds(base, 16)], mask=m)
            plsc.store_scatter(acci, [pos], ci[pl.ds(base, 16)], mask=m)

        pairs = []
        for c in range(8):
            k = accv[pl.ds(c * 16, 16)]
            v = acci[pl.ds(c * 16, 16)]
            sk, sv = plsc.sort_key_val(k, v)
            pairs.append([(sk, sv)])
        while len(pairs) > 1:
            nxt = []
            for i in range(0, len(pairs), 2):
                nxt.append(_merge(pairs[i], pairs[i + 1]))
            pairs = nxt
        srt = pairs[0]

        rbase = row * N
        for c in range(8):
            sk, sv = srt[c]
            accv[pl.ds(c * 16, 16)] = sk
            acci[pl.ds(c * 16, 16)] = sv
            gidx[pl.ds(c * 16, 16)] = sv + rbase

        pltpu.async_copy(theta_ref.at[gidx], thb, sem).wait()
        pltpu.sync_copy(accv, sd_ref.at[row])
        pltpu.sync_copy(acci, ix_ref.at[row])
        pltpu.sync_copy(thb, th_ref.at[row])
        return 0

    lax.fori_loop(0, RPW, do_row, 0)


def _sc_topk(dist2, theta_flat):
    f32 = jnp.float32
    i32 = jnp.int32
    kern = pl.kernel(
        _topk_body,
        out_type=[
            jax.ShapeDtypeStruct((ROWS, K), f32),
            jax.ShapeDtypeStruct((ROWS, K), i32),
            jax.ShapeDtypeStruct((ROWS, K), f32),
        ],
        mesh=plsc.VectorSubcoreMesh(
            core_axis_name="c", subcore_axis_name="s",
            num_cores=NC, num_subcores=NS),
        compiler_params=pltpu.CompilerParams(needs_layout_passes=False),
        scratch_types=[
            pltpu.VMEM((N,), f32),
            pltpu.VMEM((N,), f32),
            pltpu.VMEM((N,), i32),
            pltpu.VMEM((4096,), i32),
            pltpu.VMEM((256,), i32),
            pltpu.SMEM((16,), i32),
            pltpu.VMEM((K,), f32),
            pltpu.VMEM((K,), i32),
            pltpu.VMEM((K,), i32),
            pltpu.VMEM((K,), f32),
            pltpu.SemaphoreType.DMA,
        ],
    )
    return kern(dist2, theta_flat)


def _mlp_body(sd_ref, th_ref, sc_ref, w1a_ref, w1b_ref, w1c_ref, b1_ref,
              w2_ref, b2_ref, w3_ref, b3_ref, w4_ref, b4_ref, nw_ref, nb_ref,
              out_ref):
    sd = sd_ref[...]
    mx = sd[:, K - 1:K]
    sdn = sd / mx
    x = (jnp.dot(sdn, w1a_ref[...], preferred_element_type=jnp.float32)
         + jnp.dot(th_ref[...], w1b_ref[...], preferred_element_type=jnp.float32)
         + sc_ref[...] * w1c_ref[...] + b1_ref[...])
    emb = jnp.maximum(x, 0.0)
    h = jnp.maximum(
        jnp.dot(emb, w2_ref[...], preferred_element_type=jnp.float32)
        + b2_ref[...], 0.0)
    hr = h.reshape(ROWS // 16, 16, 512)
    mean = jnp.mean(hr, axis=1, keepdims=True)
    var = jnp.mean((hr - mean) * (hr - mean), axis=1, keepdims=True)
    hn = (hr - mean) * lax.rsqrt(var + EPS)
    hn = hn * nw_ref[...].reshape(1, 1, 512) + nb_ref[...].reshape(1, 1, 512)
    h2 = hn.reshape(ROWS, 512)
    emb2 = jnp.maximum(
        jnp.dot(h2, w3_ref[...], preferred_element_type=jnp.float32)
        + b3_ref[...], 0.0)
    out_ref[...] = (jnp.dot(emb2, w4_ref[...], preferred_element_type=jnp.float32)
                    + b4_ref[...] - sdn)


def _tc_mlp(sd, th, scale2, w1a, w1b, w1c, b1, w2t, b2, w3t, b3, w4t, b4,
            nw, nb):
    return pl.pallas_call(
        _mlp_body,
        out_shape=jax.ShapeDtypeStruct((ROWS, K), jnp.float32),
    )(sd, th, scale2, w1a, w1b, w1c, b1, w2t, b2, w3t, b3, w4t, b4, nw, nb)


def _scat_body(vals_ref, idx_ref, out_ref, rowbuf, ivb, vvb):
    pen = jnp.full((16,), PENALTY, jnp.float32)
    wid = lax.axis_index("c") * NS + lax.axis_index("s")

    def fill(i, _):
        rowbuf[pl.ds(i * 16, 16)] = pen
        return 0

    lax.fori_loop(0, NCHUNK, fill, 0)

    def do_row(r, _):
        row = wid * RPW + r
        pltpu.sync_copy(idx_ref.at[row], ivb)
        pltpu.sync_copy(vals_ref.at[row], vvb)
        for c in range(8):
            iv = ivb[pl.ds(c * 16, 16)]
            vv = vvb[pl.ds(c * 16, 16)]
            plsc.store_scatter(rowbuf, [iv], vv)
        pltpu.sync_copy(rowbuf, out_ref.at[row >> 4, row & 15])
        for c in range(8):
            iv = ivb[pl.ds(c * 16, 16)]
            plsc.store_scatter(rowbuf, [iv], pen)
        return 0

    lax.fori_loop(0, RPW, do_row, 0)


def _sc_scatter(vals, idx):
    kern = pl.kernel(
        _scat_body,
        out_type=jax.ShapeDtypeStruct((ROWS // 16, 16, N), jnp.float32),
        mesh=plsc.VectorSubcoreMesh(
            core_axis_name="c", subcore_axis_name="s",
            num_cores=NC, num_subcores=NS),
        compiler_params=pltpu.CompilerParams(needs_layout_passes=False),
        scratch_types=[
            pltpu.VMEM((N,), jnp.float32),
            pltpu.VMEM((K,), jnp.int32),
            pltpu.VMEM((K,), jnp.float32),
        ],
    )
    return kern(vals, idx)


@jax.jit
def kernel(dist, theta, scale, W1, b1, W2, b2, W3, b3, W4, b4, nw, nb):
    B, P, _ = dist.shape
    dist2 = dist.reshape(ROWS, N)
    theta_flat = theta.reshape(ROWS * N)
    scale2 = scale.reshape(ROWS, 1)

    sd, ix, th = _sc_topk(dist2, theta_flat)

    w1a = W1[:, :K].T
    w1b = W1[:, K:2 * K].T
    w1c = W1[:, 2 * K].reshape(1, 256)
    out = _tc_mlp(sd, th, scale2, w1a, w1b, w1c, b1.reshape(1, 256),
                  W2.T, b2.reshape(1, 512), W3.T, b3.reshape(1, 256),
                  W4.T, b4.reshape(1, 128), nw, nb)

    return _sc_scatter(out, ix)


# speedup vs baseline: 1.0616x; 1.0616x over previous
import functools

import jax
import jax.numpy as jnp
from jax import lax
from jax.experimental import pallas as pl
from jax.experimental.pallas import tpu as pltpu
from jax.experimental.pallas import tpu_sc as plsc

N = 32768
K = 128
ROWS = 512
NC, NS = 2, 16
NW = NC * NS
RPW = ROWS // NW
NCHUNK = N // 16
PENALTY = -1000000.0
EPS = 1e-5


def _cmpex(a, b):
    ak, av = a
    bk, bv = b
    swap = (bk < ak) | ((bk == ak) & (bv < av))
    lk = jnp.where(swap, bk, ak)
    lv = jnp.where(swap, bv, av)
    hk = jnp.where(swap, ak, bk)
    hv = jnp.where(swap, av, bv)
    return (lk, lv), (hk, hv)


def _bitonic_clean(seq):
    if len(seq) == 1:
        k, v = seq[0]
        sk, sv = plsc.sort_key_val(k, v)
        return [(sk, sv)]
    half = len(seq) // 2
    lo, hi = [], []
    for i in range(half):
        l, h = _cmpex(seq[i], seq[i + half])
        lo.append(l)
        hi.append(h)
    return _bitonic_clean(lo) + _bitonic_clean(hi)


def _merge(a, b):
    rb = [(lax.rev(k, (0,)), lax.rev(v, (0,))) for (k, v) in reversed(b)]
    lo, hi = [], []
    for x, y in zip(a, rb):
        l, h = _cmpex(x, y)
        lo.append(l)
        hi.append(h)
    return _bitonic_clean(lo) + _bitonic_clean(hi)


def _topk_body(dist_ref, theta_ref, sd_ref, ix_ref, th_ref,
               row_v, cv, ci, hist, totals, ctot,
               accv, acci, gidx, thb, sem):
    lane = lax.iota(jnp.int32, 16)
    lane_off = lane * 256
    ones = jnp.ones((16,), jnp.int32)
    zeros16 = jnp.zeros((16,), jnp.int32)
    wid = lax.axis_index("c") * NS + lax.axis_index("s")

    def clear_hist(i, _):
        hist[pl.ds(i * 16, 16)] = zeros16
        return 0

    def scan_hist(kp):
        for c in range(16):
            tv = hist[pl.ds(c * 16, 16)]
            for l in range(1, 16):
                tv = tv + hist[pl.ds(l * 256 + c * 16, 16)]
            totals[pl.ds(c * 16, 16)] = tv
            ctot[c] = jnp.sum(tv)

        def outer(c, st):
            cum, cc, cb = st
            t = ctot[c]
            hit = jnp.logical_and(cum < kp, cum + t >= kp)
            cc = jnp.where(hit, c, cc)
            cb = jnp.where(hit, cum, cb)
            return (cum + t, cc, cb)

        _, cc, cb = lax.fori_loop(0, 16, outer, (0, 0, 0))

        tv = totals[pl.ds(cc * 16, 16)]
        cum_v = plsc.cumsum(tv) + cb
        hitmask = cum_v >= kp
        j = plsc.all_reduce_ffs(hitmask)[0]
        tbin = cc * 16 + j
        c_less = cb + jnp.sum(jnp.where(hitmask, 0, tv))
        return tbin, c_less

    def radix_pass(shift, n, kp, acc_off, first):
        @plsc.parallel_loop(0, 256, unroll=8)
        def _(i):
            hist[pl.ds(i * 16, 16)] = zeros16

        if first:
            @plsc.parallel_loop(0, NCHUNK, unroll=8)
            def _(i):
                v = row_v[pl.ds(i * 16, 16)]
                key = lax.bitcast_convert_type(v, jnp.int32)
                byte = (key >> shift) & 255
                plsc.addupdate_scatter(hist, [lane_off + byte], ones)
            nch = NCHUNK
        else:
            def hbody(i, _):
                base = i * 16
                v = cv[pl.ds(base, 16)]
                key = lax.bitcast_convert_type(v, jnp.int32)
                byte = (key >> shift) & 255
                m = (base + lane) < n
                plsc.addupdate_scatter(hist, [lane_off + byte], ones, mask=m)
                return 0

            nch = pl.cdiv(n, 16)
            lax.fori_loop(0, nch, hbody, 0)

        tbin, c_less = scan_hist(kp)
        tsplat = jnp.full((16,), tbin, jnp.int32)

        def cstep(i, offa, offe, v, iv, m_lt, m_eq):
            r_lt = plsc.cumsum(jnp.where(m_lt, 1, 0))
            r_eq = plsc.cumsum(jnp.where(m_eq, 1, 0))
            pos_a = offa + r_lt - 1
            pos_e = offe + r_eq - 1
            plsc.store_scatter(accv, [pos_a], v, mask=m_lt)
            plsc.store_scatter(acci, [pos_a], iv, mask=m_lt)
            if first:
                plsc.store_scatter(cv, [pos_e], v, mask=m_eq)
                plsc.store_scatter(ci, [pos_e], iv, mask=m_eq)
            else:
                plsc.store_scatter(cv, [pos_e], v, mask=m_eq)
                plsc.store_scatter(ci, [pos_e], iv, mask=m_eq)
            offa = offa + plsc.all_reduce_population_count(m_lt)
            offe = offe + plsc.all_reduce_population_count(m_eq)
            return offa, offe

        offa0 = jnp.full((16,), acc_off, jnp.int32)
        if first:
            @plsc.parallel_loop(0, NCHUNK, unroll=4, carry=(offa0, zeros16))
            def offs_out(i, offs):
                offa, offe = offs
                base = i * 16
                v = row_v[pl.ds(base, 16)]
                iv = base + lane
                key = lax.bitcast_convert_type(v, jnp.int32)
                byte = (key >> shift) & 255
                return cstep(i, offa, offe, v, iv, byte < tsplat,
                             byte == tsplat)
            _, offe_v = offs_out
        else:
            def cbody(i, offs):
                offa, offe = offs
                base = i * 16
                gids = base + lane
                v = cv[pl.ds(base, 16)]
                iv = ci[pl.ds(base, 16)]
                mval = gids < n
                key = lax.bitcast_convert_type(v, jnp.int32)
                byte = (key >> shift) & 255
                m_lt = jnp.logical_and(byte < tsplat, mval)
                m_eq = jnp.logical_and(byte == tsplat, mval)
                return cstep(i, offa, offe, v, iv, m_lt, m_eq)

            _, offe_v = lax.fori_loop(0, nch, cbody, (offa0, zeros16))
        return kp - c_less, offe_v[0], acc_off + c_less

    def do_row(r, _):
        row = wid * RPW + r
        abase = r * K
        pltpu.sync_copy(dist_ref.at[row], row_v)

        kp, n, off = radix_pass(24, N, 128, abase, True)
        kp, n, off = radix_pass(16, n, kp, off, False)
        kp, n, off = radix_pass(8, n, kp, off, False)
        kp, n, off = radix_pass(0, n, kp, off, False)

        offv = jnp.full((16,), off, jnp.int32)
        for c in range(8):
            base = c * 16
            m = (base + lane) < kp
            pos = offv + base + lane
            plsc.store_scatter(accv, [pos], cv[pl.ds(base, 16)], mask=m)
            plsc.store_scatter(acci, [pos], ci[pl.ds(base, 16)], mask=m)

        pairs = []
        for c in range(8):
            k = accv[pl.ds(abase + c * 16, 16)]
            v = acci[pl.ds(abase + c * 16, 16)]
            sk, sv = plsc.sort_key_val(k, v)
            pairs.append([(sk, sv)])
        while len(pairs) > 1:
            nxt = []
            for i in range(0, len(pairs), 2):
                nxt.append(_merge(pairs[i], pairs[i + 1]))
            pairs = nxt
        srt = pairs[0]

        rbase = row * N
        for c in range(8):
            sk, sv = srt[c]
            accv[pl.ds(abase + c * 16, 16)] = sk
            acci[pl.ds(abase + c * 16, 16)] = sv
            gidx[pl.ds(abase + c * 16, 16)] = sv + rbase
        return 0

    lax.fori_loop(0, RPW, do_row, 0)

    pltpu.async_copy(theta_ref.at[gidx], thb, sem).wait()
    pltpu.sync_copy(accv, sd_ref.at[wid])
    pltpu.sync_copy(acci, ix_ref.at[wid])
    pltpu.sync_copy(thb, th_ref.at[wid])


def _sc_topk(dist2, theta_flat):
    f32 = jnp.float32
    i32 = jnp.int32
    kern = pl.kernel(
        _topk_body,
        out_type=[
            jax.ShapeDtypeStruct((NW, RPW * K), f32),
            jax.ShapeDtypeStruct((NW, RPW * K), i32),
            jax.ShapeDtypeStruct((NW, RPW * K), f32),
        ],
        mesh=plsc.VectorSubcoreMesh(
            core_axis_name="c", subcore_axis_name="s",
            num_cores=NC, num_subcores=NS),
        compiler_params=pltpu.CompilerParams(needs_layout_passes=False),
        scratch_types=[
            pltpu.VMEM((N,), f32),
            pltpu.VMEM((N,), f32),
            pltpu.VMEM((N,), i32),
            pltpu.VMEM((4096,), i32),
            pltpu.VMEM((256,), i32),
            pltpu.SMEM((16,), i32),
            pltpu.VMEM((RPW * K,), f32),
            pltpu.VMEM((RPW * K,), i32),
            pltpu.VMEM((RPW * K,), i32),
            pltpu.VMEM((RPW * K,), f32),
            pltpu.SemaphoreType.DMA,
        ],
    )
    return kern(dist2, theta_flat)


def _mlp_body(sd_ref, th_ref, sc_ref, w1a_ref, w1b_ref, w1c_ref, b1_ref,
              w2_ref, b2_ref, w3_ref, b3_ref, w4_ref, b4_ref, nw_ref, nb_ref,
              out_ref):
    sd = sd_ref[...]
    mx = sd[:, K - 1:K]
    sdn = sd / mx
    x = (jnp.dot(sdn, w1a_ref[...], preferred_element_type=jnp.float32)
         + jnp.dot(th_ref[...], w1b_ref[...], preferred_element_type=jnp.float32)
         + sc_ref[...] * w1c_ref[...] + b1_ref[...])
    emb = jnp.maximum(x, 0.0)
    h = jnp.maximum(
        jnp.dot(emb, w2_ref[...], preferred_element_type=jnp.float32)
        + b2_ref[...], 0.0)
    hr = h.reshape(ROWS // 16, 16, 512)
    mean = jnp.mean(hr, axis=1, keepdims=True)
    var = jnp.mean((hr - mean) * (hr - mean), axis=1, keepdims=True)
    hn = (hr - mean) * lax.rsqrt(var + EPS)
    hn = hn * nw_ref[...].reshape(1, 1, 512) + nb_ref[...].reshape(1, 1, 512)
    h2 = hn.reshape(ROWS, 512)
    emb2 = jnp.maximum(
        jnp.dot(h2, w3_ref[...], preferred_element_type=jnp.float32)
        + b3_ref[...], 0.0)
    out_ref[...] = (jnp.dot(emb2, w4_ref[...], preferred_element_type=jnp.float32)
                    + b4_ref[...] - sdn)


def _tc_mlp(sd, th, scale2, w1a, w1b, w1c, b1, w2t, b2, w3t, b3, w4t, b4,
            nw, nb):
    return pl.pallas_call(
        _mlp_body,
        out_shape=jax.ShapeDtypeStruct((ROWS, K), jnp.float32),
    )(sd, th, scale2, w1a, w1b, w1c, b1, w2t, b2, w3t, b3, w4t, b4, nw, nb)


def _scat_body(vals_ref, idx_ref, out_ref, rowbuf, ivb, vvb):
    pen = jnp.full((16,), PENALTY, jnp.float32)
    wid = lax.axis_index("c") * NS + lax.axis_index("s")

    def fill(i, _):
        rowbuf[pl.ds(i * 16, 16)] = pen
        return 0

    lax.fori_loop(0, NCHUNK, fill, 0)

    def do_row(r, _):
        row = wid * RPW + r
        pltpu.sync_copy(idx_ref.at[row], ivb)
        pltpu.sync_copy(vals_ref.at[row], vvb)
        for c in range(8):
            iv = ivb[pl.ds(c * 16, 16)]
            vv = vvb[pl.ds(c * 16, 16)]
            plsc.store_scatter(rowbuf, [iv], vv)
        pltpu.sync_copy(rowbuf, out_ref.at[row >> 4, row & 15])
        for c in range(8):
            iv = ivb[pl.ds(c * 16, 16)]
            plsc.store_scatter(rowbuf, [iv], pen)
        return 0

    lax.fori_loop(0, RPW, do_row, 0)


def _sc_scatter(vals, idx):
    kern = pl.kernel(
        _scat_body,
        out_type=jax.ShapeDtypeStruct((ROWS // 16, 16, N), jnp.float32),
        mesh=plsc.VectorSubcoreMesh(
            core_axis_name="c", subcore_axis_name="s",
            num_cores=NC, num_subcores=NS),
        compiler_params=pltpu.CompilerParams(needs_layout_passes=False),
        scratch_types=[
            pltpu.VMEM((N,), jnp.float32),
            pltpu.VMEM((K,), jnp.int32),
            pltpu.VMEM((K,), jnp.float32),
        ],
    )
    return kern(vals, idx)


@jax.jit
def kernel(dist, theta, scale, W1, b1, W2, b2, W3, b3, W4, b4, nw, nb):
    B, P, _ = dist.shape
    dist2 = dist.reshape(ROWS, N)
    theta_flat = theta.reshape(ROWS * N)
    scale2 = scale.reshape(ROWS, 1)

    sd, ix, th = _sc_topk(dist2, theta_flat)
    sd = sd.reshape(ROWS, K)
    ix = ix.reshape(ROWS, K)
    th = th.reshape(ROWS, K)

    w1a = W1[:, :K].T
    w1b = W1[:, K:2 * K].T
    w1c = W1[:, 2 * K].reshape(1, 256)
    out = _tc_mlp(sd, th, scale2, w1a, w1b, w1c, b1.reshape(1, 256),
                  W2.T, b2.reshape(1, 512), W3.T, b3.reshape(1, 256),
                  W4.T, b4.reshape(1, 128), nw, nb)

    return _sc_scatter(out, ix)


# speedup vs baseline: 1.1199x; 1.0549x over previous
import functools

import jax
import jax.numpy as jnp
from jax import lax
from jax.experimental import pallas as pl
from jax.experimental.pallas import tpu as pltpu
from jax.experimental.pallas import tpu_sc as plsc

N = 32768
K = 128
ROWS = 512
NC, NS = 2, 16
NW = NC * NS
RPW = ROWS // NW
NCHUNK = N // 16
PENALTY = -1000000.0
EPS = 1e-5


def _cmpex(a, b):
    ak, av = a
    bk, bv = b
    swap = (bk < ak) | ((bk == ak) & (bv < av))
    lk = jnp.where(swap, bk, ak)
    lv = jnp.where(swap, bv, av)
    hk = jnp.where(swap, ak, bk)
    hv = jnp.where(swap, av, bv)
    return (lk, lv), (hk, hv)


def _bitonic_clean(seq):
    if len(seq) == 1:
        k, v = seq[0]
        sk, sv = plsc.sort_key_val(k, v)
        return [(sk, sv)]
    half = len(seq) // 2
    lo, hi = [], []
    for i in range(half):
        l, h = _cmpex(seq[i], seq[i + half])
        lo.append(l)
        hi.append(h)
    return _bitonic_clean(lo) + _bitonic_clean(hi)


def _merge(a, b):
    rb = [(lax.rev(k, (0,)), lax.rev(v, (0,))) for (k, v) in reversed(b)]
    lo, hi = [], []
    for x, y in zip(a, rb):
        l, h = _cmpex(x, y)
        lo.append(l)
        hi.append(h)
    return _bitonic_clean(lo) + _bitonic_clean(hi)


def _topk_body(dist_ref, theta_ref, sd_ref, ix_ref, th_ref,
               row_v, cv, ci, hist, totals, ctot,
               accv, acci, gidx, thb, sem, sem2):
    lane = lax.iota(jnp.int32, 16)
    lane_off = lane * 256
    ones = jnp.ones((16,), jnp.int32)
    zeros16 = jnp.zeros((16,), jnp.int32)
    wid = lax.axis_index("c") * NS + lax.axis_index("s")

    def clear_hist(i, _):
        hist[pl.ds(i * 16, 16)] = zeros16
        return 0

    def scan_hist(kp):
        for c in range(16):
            tv = hist[pl.ds(c * 16, 16)]
            for l in range(1, 16):
                tv = tv + hist[pl.ds(l * 256 + c * 16, 16)]
            totals[pl.ds(c * 16, 16)] = tv
            ctot[c] = jnp.sum(tv)

        def outer(c, st):
            cum, cc, cb = st
            t = ctot[c]
            hit = jnp.logical_and(cum < kp, cum + t >= kp)
            cc = jnp.where(hit, c, cc)
            cb = jnp.where(hit, cum, cb)
            return (cum + t, cc, cb)

        _, cc, cb = lax.fori_loop(0, 16, outer, (0, 0, 0))

        tv = totals[pl.ds(cc * 16, 16)]
        cum_v = plsc.cumsum(tv) + cb
        hitmask = cum_v >= kp
        j = plsc.all_reduce_ffs(hitmask)[0]
        tbin = cc * 16 + j
        c_less = cb + jnp.sum(jnp.where(hitmask, 0, tv))
        return tbin, c_less

    def radix_pass(shift, n, kp, acc_off, first):
        @plsc.parallel_loop(0, 256, unroll=8)
        def _(i):
            hist[pl.ds(i * 16, 16)] = zeros16

        if first:
            @plsc.parallel_loop(0, NCHUNK, unroll=8)
            def _(i):
                v = row_v[pl.ds(i * 16, 16)]
                key = lax.bitcast_convert_type(v, jnp.int32)
                byte = (key >> shift) & 255
                plsc.addupdate_scatter(hist, [lane_off + byte], ones)
            nch = NCHUNK
        else:
            def hbody(i, _):
                base = i * 16
                v = cv[pl.ds(base, 16)]
                key = lax.bitcast_convert_type(v, jnp.int32)
                byte = (key >> shift) & 255
                m = (base + lane) < n
                plsc.addupdate_scatter(hist, [lane_off + byte], ones, mask=m)
                return 0

            nch = pl.cdiv(n, 16)
            lax.fori_loop(0, nch, hbody, 0)

        tbin, c_less = scan_hist(kp)
        tsplat = jnp.full((16,), tbin, jnp.int32)

        def cstep(i, offa, offe, v, iv, m_lt, m_eq):
            r_lt = plsc.cumsum(jnp.where(m_lt, 1, 0))
            r_eq = plsc.cumsum(jnp.where(m_eq, 1, 0))
            pos_a = offa + r_lt - 1
            pos_e = offe + r_eq - 1
            plsc.store_scatter(accv, [pos_a], v, mask=m_lt)
            plsc.store_scatter(acci, [pos_a], iv, mask=m_lt)
            if first:
                plsc.store_scatter(cv, [pos_e], v, mask=m_eq)
                plsc.store_scatter(ci, [pos_e], iv, mask=m_eq)
            else:
                plsc.store_scatter(cv, [pos_e], v, mask=m_eq)
                plsc.store_scatter(ci, [pos_e], iv, mask=m_eq)
            offa = offa + plsc.all_reduce_population_count(m_lt)
            offe = offe + plsc.all_reduce_population_count(m_eq)
            return offa, offe

        offa0 = jnp.full((16,), acc_off, jnp.int32)
        if first:
            @plsc.parallel_loop(0, NCHUNK, unroll=4, carry=(offa0, zeros16))
            def offs_out(i, offs):
                offa, offe = offs
                base = i * 16
                v = row_v[pl.ds(base, 16)]
                iv = base + lane
                key = lax.bitcast_convert_type(v, jnp.int32)
                byte = (key >> shift) & 255
                return cstep(i, offa, offe, v, iv, byte < tsplat,
                             byte == tsplat)
            _, offe_v = offs_out
        else:
            def cbody(i, offs):
                offa, offe = offs
                base = i * 16
                gids = base + lane
                v = cv[pl.ds(base, 16)]
                iv = ci[pl.ds(base, 16)]
                mval = gids < n
                key = lax.bitcast_convert_type(v, jnp.int32)
                byte = (key >> shift) & 255
                m_lt = jnp.logical_and(byte < tsplat, mval)
                m_eq = jnp.logical_and(byte == tsplat, mval)
                return cstep(i, offa, offe, v, iv, m_lt, m_eq)

            _, offe_v = lax.fori_loop(0, nch, cbody, (offa0, zeros16))
        return kp - c_less, offe_v[0], acc_off + c_less

    def do_row(r, _):
        row = wid * RPW + r
        abase = r * K

        @pl.when(r > 0)
        def _():
            pltpu.make_async_copy(dist_ref.at[row], row_v, sem2).wait()

        kp, n, off = radix_pass(24, N, 128, abase, True)

        @pl.when(r < RPW - 1)
        def _():
            pltpu.async_copy(dist_ref.at[row + 1], row_v, sem2)

        kp, n, off = radix_pass(16, n, kp, off, False)
        kp, n, off = radix_pass(8, n, kp, off, False)
        kp, n, off = radix_pass(0, n, kp, off, False)

        offv = jnp.full((16,), off, jnp.int32)
        for c in range(8):
            base = c * 16
            m = (base + lane) < kp
            pos = offv + base + lane
            plsc.store_scatter(accv, [pos], cv[pl.ds(base, 16)], mask=m)
            plsc.store_scatter(acci, [pos], ci[pl.ds(base, 16)], mask=m)

        pairs = []
        for c in range(8):
            k = accv[pl.ds(abase + c * 16, 16)]
            v = acci[pl.ds(abase + c * 16, 16)]
            sk, sv = plsc.sort_key_val(k, v)
            pairs.append([(sk, sv)])
        while len(pairs) > 1:
            nxt = []
            for i in range(0, len(pairs), 2):
                nxt.append(_merge(pairs[i], pairs[i + 1]))
            pairs = nxt
        srt = pairs[0]

        rbase = row * N
        for c in range(8):
            sk, sv = srt[c]
            accv[pl.ds(abase + c * 16, 16)] = sk
            acci[pl.ds(abase + c * 16, 16)] = sv
            gidx[pl.ds(abase + c * 16, 16)] = sv + rbase
        return 0

    pltpu.sync_copy(dist_ref.at[wid * RPW], row_v)
    lax.fori_loop(0, RPW, do_row, 0)

    pltpu.async_copy(theta_ref.at[gidx], thb, sem).wait()
    pltpu.sync_copy(accv, sd_ref.at[wid])
    pltpu.sync_copy(acci, ix_ref.at[wid])
    pltpu.sync_copy(thb, th_ref.at[wid])


def _sc_topk(dist2, theta_flat):
    f32 = jnp.float32
    i32 = jnp.int32
    kern = pl.kernel(
        _topk_body,
        out_type=[
            jax.ShapeDtypeStruct((NW, RPW * K), f32),
            jax.ShapeDtypeStruct((NW, RPW * K), i32),
            jax.ShapeDtypeStruct((NW, RPW * K), f32),
        ],
        mesh=plsc.VectorSubcoreMesh(
            core_axis_name="c", subcore_axis_name="s",
            num_cores=NC, num_subcores=NS),
        compiler_params=pltpu.CompilerParams(needs_layout_passes=False),
        scratch_types=[
            pltpu.VMEM((N,), f32),
            pltpu.VMEM((N,), f32),
            pltpu.VMEM((N,), i32),
            pltpu.VMEM((4096,), i32),
            pltpu.VMEM((256,), i32),
            pltpu.SMEM((16,), i32),
            pltpu.VMEM((RPW * K,), f32),
            pltpu.VMEM((RPW * K,), i32),
            pltpu.VMEM((RPW * K,), i32),
            pltpu.VMEM((RPW * K,), f32),
            pltpu.SemaphoreType.DMA,
            pltpu.SemaphoreType.DMA,
        ],
    )
    return kern(dist2, theta_flat)


def _mlp_body(sd_ref, th_ref, sc_ref, w1a_ref, w1b_ref, w1c_ref, b1_ref,
              w2_ref, b2_ref, w3_ref, b3_ref, w4_ref, b4_ref, nw_ref, nb_ref,
              out_ref):
    sd = sd_ref[...]
    mx = sd[:, K - 1:K]
    sdn = sd / mx
    x = (jnp.dot(sdn, w1a_ref[...], preferred_element_type=jnp.float32)
         + jnp.dot(th_ref[...], w1b_ref[...], preferred_element_type=jnp.float32)
         + sc_ref[...] * w1c_ref[...] + b1_ref[...])
    emb = jnp.maximum(x, 0.0)
    h = jnp.maximum(
        jnp.dot(emb, w2_ref[...], preferred_element_type=jnp.float32)
        + b2_ref[...], 0.0)
    hr = h.reshape(ROWS // 16, 16, 512)
    mean = jnp.mean(hr, axis=1, keepdims=True)
    var = jnp.mean((hr - mean) * (hr - mean), axis=1, keepdims=True)
    hn = (hr - mean) * lax.rsqrt(var + EPS)
    hn = hn * nw_ref[...].reshape(1, 1, 512) + nb_ref[...].reshape(1, 1, 512)
    h2 = hn.reshape(ROWS, 512)
    emb2 = jnp.maximum(
        jnp.dot(h2, w3_ref[...], preferred_element_type=jnp.float32)
        + b3_ref[...], 0.0)
    out_ref[...] = (jnp.dot(emb2, w4_ref[...], preferred_element_type=jnp.float32)
                    + b4_ref[...] - sdn)


def _tc_mlp(sd, th, scale2, w1a, w1b, w1c, b1, w2t, b2, w3t, b3, w4t, b4,
            nw, nb):
    return pl.pallas_call(
        _mlp_body,
        out_shape=jax.ShapeDtypeStruct((ROWS, K), jnp.float32),
    )(sd, th, scale2, w1a, w1b, w1c, b1, w2t, b2, w3t, b3, w4t, b4, nw, nb)


def _scat_body(vals_ref, idx_ref, out_ref, rowbuf, ivb, vvb):
    pen = jnp.full((16,), PENALTY, jnp.float32)
    wid = lax.axis_index("c") * NS + lax.axis_index("s")

    def fill(i, _):
        rowbuf[pl.ds(i * 16, 16)] = pen
        return 0

    lax.fori_loop(0, NCHUNK, fill, 0)

    def do_row(r, _):
        row = wid * RPW + r
        pltpu.sync_copy(idx_ref.at[row], ivb)
        pltpu.sync_copy(vals_ref.at[row], vvb)
        for c in range(8):
            iv = ivb[pl.ds(c * 16, 16)]
            vv = vvb[pl.ds(c * 16, 16)]
            plsc.store_scatter(rowbuf, [iv], vv)
        pltpu.sync_copy(rowbuf, out_ref.at[row >> 4, row & 15])
        for c in range(8):
            iv = ivb[pl.ds(c * 16, 16)]
            plsc.store_scatter(rowbuf, [iv], pen)
        return 0

    lax.fori_loop(0, RPW, do_row, 0)


def _sc_scatter(vals, idx):
    kern = pl.kernel(
        _scat_body,
        out_type=jax.ShapeDtypeStruct((ROWS // 16, 16, N), jnp.float32),
        mesh=plsc.VectorSubcoreMesh(
            core_axis_name="c", subcore_axis_name="s",
            num_cores=NC, num_subcores=NS),
        compiler_params=pltpu.CompilerParams(needs_layout_passes=False),
        scratch_types=[
            pltpu.VMEM((N,), jnp.float32),
            pltpu.VMEM((K,), jnp.int32),
            pltpu.VMEM((K,), jnp.float32),
        ],
    )
    return kern(vals, idx)


@jax.jit
def kernel(dist, theta, scale, W1, b1, W2, b2, W3, b3, W4, b4, nw, nb):
    B, P, _ = dist.shape
    dist2 = dist.reshape(ROWS, N)
    theta_flat = theta.reshape(ROWS * N)
    scale2 = scale.reshape(ROWS, 1)

    sd, ix, th = _sc_topk(dist2, theta_flat)
    sd = sd.reshape(ROWS, K)
    ix = ix.reshape(ROWS, K)
    th = th.reshape(ROWS, K)

    w1a = W1[:, :K].T
    w1b = W1[:, K:2 * K].T
    w1c = W1[:, 2 * K].reshape(1, 256)
    out = _tc_mlp(sd, th, scale2, w1a, w1b, w1c, b1.reshape(1, 256),
                  W2.T, b2.reshape(1, 512), W3.T, b3.reshape(1, 256),
                  W4.T, b4.reshape(1, 128), nw, nb)

    return _sc_scatter(out, ix)


# speedup vs baseline: 1.1206x; 1.0006x over previous
import jax
import jax.numpy as jnp
from jax import lax
from jax.experimental import pallas as pl
from jax.experimental.pallas import tpu as pltpu
from jax.experimental.pallas import tpu_sc as plsc

N = 32768
K = 128
ROWS = 512
NC, NS = 2, 16
NW = NC * NS
RPW = ROWS // NW
NCHUNK = N // 16
PENALTY = -1000000.0
EPS = 1e-5


def _cmpex(a, b):
    ak, av = a
    bk, bv = b
    swap = (bk < ak) | ((bk == ak) & (bv < av))
    lk = jnp.where(swap, bk, ak)
    lv = jnp.where(swap, bv, av)
    hk = jnp.where(swap, ak, bk)
    hv = jnp.where(swap, av, bv)
    return (lk, lv), (hk, hv)


def _bitonic_clean(seq):
    if len(seq) == 1:
        k, v = seq[0]
        sk, sv = plsc.sort_key_val(k, v)
        return [(sk, sv)]
    half = len(seq) // 2
    lo, hi = [], []
    for i in range(half):
        l, h = _cmpex(seq[i], seq[i + half])
        lo.append(l)
        hi.append(h)
    return _bitonic_clean(lo) + _bitonic_clean(hi)


def _merge(a, b):
    rb = [(lax.rev(k, (0,)), lax.rev(v, (0,))) for (k, v) in reversed(b)]
    lo, hi = [], []
    for x, y in zip(a, rb):
        l, h = _cmpex(x, y)
        lo.append(l)
        hi.append(h)
    return _bitonic_clean(lo) + _bitonic_clean(hi)


def _topk_body(dist_ref, theta_ref, sd_ref, ix_ref, th_ref,
               row_v, cv, ci, hist, totals, ctot,
               accv, acci, gidx, thb, sem, sem2):
    lane = lax.iota(jnp.int32, 16)
    lane_off = lane * 256
    ones = jnp.ones((16,), jnp.int32)
    zeros16 = jnp.zeros((16,), jnp.int32)
    wid = lax.axis_index("c") * NS + lax.axis_index("s")

    def clear_hist(i, _):
        hist[pl.ds(i * 16, 16)] = zeros16
        return 0

    def scan_hist(kp):
        for c in range(16):
            tv = hist[pl.ds(c * 16, 16)]
            for l in range(1, 16):
                tv = tv + hist[pl.ds(l * 256 + c * 16, 16)]
            totals[pl.ds(c * 16, 16)] = tv
            ctot[c] = jnp.sum(tv)

        def outer(c, st):
            cum, cc, cb = st
            t = ctot[c]
            hit = jnp.logical_and(cum < kp, cum + t >= kp)
            cc = jnp.where(hit, c, cc)
            cb = jnp.where(hit, cum, cb)
            return (cum + t, cc, cb)

        _, cc, cb = lax.fori_loop(0, 16, outer, (0, 0, 0))

        tv = totals[pl.ds(cc * 16, 16)]
        cum_v = plsc.cumsum(tv) + cb
        hitmask = cum_v >= kp
        j = plsc.all_reduce_ffs(hitmask)[0]
        tbin = cc * 16 + j
        c_less = cb + jnp.sum(jnp.where(hitmask, 0, tv))
        return tbin, c_less

    def radix_pass(shift, n, kp, acc_off, first):
        @plsc.parallel_loop(0, 256, unroll=8)
        def _(i):
            hist[pl.ds(i * 16, 16)] = zeros16

        if first:
            @plsc.parallel_loop(0, NCHUNK, unroll=8)
            def _(i):
                v = row_v[pl.ds(i * 16, 16)]
                key = lax.bitcast_convert_type(v, jnp.int32)
                byte = (key >> shift) & 255
                plsc.addupdate_scatter(hist, [lane_off + byte], ones)
            nch = NCHUNK
        else:
            def hbody(i, _):
                base = i * 16
                v = cv[pl.ds(base, 16)]
                key = lax.bitcast_convert_type(v, jnp.int32)
                byte = (key >> shift) & 255
                m = (base + lane) < n
                plsc.addupdate_scatter(hist, [lane_off + byte], ones, mask=m)
                return 0

            nch = pl.cdiv(n, 16)
            lax.fori_loop(0, nch, hbody, 0)

        tbin, c_less = scan_hist(kp)
        tsplat = jnp.full((16,), tbin, jnp.int32)

        def cstep(i, offa, offe, v, iv, m_lt, m_eq):
            r_lt = plsc.cumsum(jnp.where(m_lt, 1, 0))
            r_eq = plsc.cumsum(jnp.where(m_eq, 1, 0))
            pos_a = offa + r_lt - 1
            pos_e = offe + r_eq - 1
            plsc.store_scatter(accv, [pos_a], v, mask=m_lt)
            plsc.store_scatter(acci, [pos_a], iv, mask=m_lt)
            plsc.store_scatter(cv, [pos_e], v, mask=m_eq)
            plsc.store_scatter(ci, [pos_e], iv, mask=m_eq)
            offa = offa + plsc.all_reduce_population_count(m_lt)
            offe = offe + plsc.all_reduce_population_count(m_eq)
            return offa, offe

        offa0 = jnp.full((16,), acc_off, jnp.int32)
        if first:
            @plsc.parallel_loop(0, NCHUNK, unroll=4, carry=(offa0, zeros16))
            def offs_out(i, offs):
                offa, offe = offs
                base = i * 16
                v = row_v[pl.ds(base, 16)]
                iv = base + lane
                key = lax.bitcast_convert_type(v, jnp.int32)
                byte = (key >> shift) & 255
                return cstep(i, offa, offe, v, iv, byte < tsplat,
                             byte == tsplat)
            _, offe_v = offs_out
        else:
            def cbody(i, offs):
                offa, offe = offs
                base = i * 16
                gids = base + lane
                v = cv[pl.ds(base, 16)]
                iv = ci[pl.ds(base, 16)]
                mval = gids < n
                key = lax.bitcast_convert_type(v, jnp.int32)
                byte = (key >> shift) & 255
                m_lt = jnp.logical_and(byte < tsplat, mval)
                m_eq = jnp.logical_and(byte == tsplat, mval)
                return cstep(i, offa, offe, v, iv, m_lt, m_eq)

            _, offe_v = lax.fori_loop(0, nch, cbody, (offa0, zeros16))
        return kp - c_less, offe_v[0], acc_off + c_less

    def do_row(r, _):
        row = wid * RPW + r
        abase = r * K

        @pl.when(r > 0)
        def _():
            pltpu.make_async_copy(dist_ref.at[row], row_v, sem2).wait()

        kp, n, off = radix_pass(24, N, 128, abase, True)

        @pl.when(r < RPW - 1)
        def _():
            pltpu.async_copy(dist_ref.at[row + 1], row_v, sem2)

        kp, n, off = radix_pass(16, n, kp, off, False)
        kp, n, off = radix_pass(8, n, kp, off, False)
        kp, n, off = radix_pass(0, n, kp, off, False)

        offv = jnp.full((16,), off, jnp.int32)
        for c in range(8):
            base = c * 16
            m = (base + lane) < kp
            pos = offv + base + lane
            plsc.store_scatter(accv, [pos], cv[pl.ds(base, 16)], mask=m)
            plsc.store_scatter(acci, [pos], ci[pl.ds(base, 16)], mask=m)

        pairs = []
        for c in range(8):
            k = accv[pl.ds(abase + c * 16, 16)]
            v = acci[pl.ds(abase + c * 16, 16)]
            sk, sv = plsc.sort_key_val(k, v)
            pairs.append([(sk, sv)])
        while len(pairs) > 1:
            nxt = []
            for i in range(0, len(pairs), 2):
                nxt.append(_merge(pairs[i], pairs[i + 1]))
            pairs = nxt
        srt = pairs[0]

        rbase = row * N
        for c in range(8):
            sk, sv = srt[c]
            accv[pl.ds(abase + c * 16, 16)] = sk
            acci[pl.ds(abase + c * 16, 16)] = sv
            gidx[pl.ds(abase + c * 16, 16)] = sv + rbase
        return 0

    pltpu.sync_copy(dist_ref.at[wid * RPW], row_v)
    lax.fori_loop(0, RPW, do_row, 0)

    pltpu.async_copy(theta_ref.at[gidx], thb, sem).wait()
    pltpu.sync_copy(accv, sd_ref.at[wid])
    pltpu.sync_copy(acci, ix_ref.at[wid])
    pltpu.sync_copy(thb, th_ref.at[wid])


def _sc_topk(dist2, theta_flat):
    f32 = jnp.float32
    i32 = jnp.int32
    kern = pl.kernel(
        _topk_body,
        out_type=[
            jax.ShapeDtypeStruct((NW, RPW * K), f32),
            jax.ShapeDtypeStruct((NW, RPW * K), i32),
            jax.ShapeDtypeStruct((NW, RPW * K), f32),
        ],
        mesh=plsc.VectorSubcoreMesh(
            core_axis_name="c", subcore_axis_name="s",
            num_cores=NC, num_subcores=NS),
        compiler_params=pltpu.CompilerParams(needs_layout_passes=False),
        scratch_types=[
            pltpu.VMEM((N,), f32),
            pltpu.VMEM((N,), f32),
            pltpu.VMEM((N,), i32),
            pltpu.VMEM((4096,), i32),
            pltpu.VMEM((256,), i32),
            pltpu.SMEM((16,), i32),
            pltpu.VMEM((RPW * K,), f32),
            pltpu.VMEM((RPW * K,), i32),
            pltpu.VMEM((RPW * K,), i32),
            pltpu.VMEM((RPW * K,), f32),
            pltpu.SemaphoreType.DMA,
            pltpu.SemaphoreType.DMA,
        ],
    )
    return kern(dist2, theta_flat)


def _mlp_body(sd_ref, th_ref, sc_ref, w1a_ref, w1b_ref, w1c_ref, b1_ref,
              w2_ref, b2_ref, w3_ref, b3_ref, w4_ref, b4_ref, nw_ref, nb_ref,
              out_ref):
    sd = sd_ref[...]
    mx = sd[:, K - 1:K]
    sdn = sd / mx
    x = (jnp.dot(sdn, w1a_ref[...], preferred_element_type=jnp.float32)
         + jnp.dot(th_ref[...], w1b_ref[...], preferred_element_type=jnp.float32)
         + sc_ref[...] * w1c_ref[...] + b1_ref[...])
    emb = jnp.maximum(x, 0.0)
    h = jnp.maximum(
        jnp.dot(emb, w2_ref[...], preferred_element_type=jnp.float32)
        + b2_ref[...], 0.0)
    hr = h.reshape(ROWS // 16, 16, 512)
    mean = jnp.mean(hr, axis=1, keepdims=True)
    var = jnp.mean((hr - mean) * (hr - mean), axis=1, keepdims=True)
    hn = (hr - mean) * lax.rsqrt(var + EPS)
    hn = hn * nw_ref[...].reshape(1, 1, 512) + nb_ref[...].reshape(1, 1, 512)
    h2 = hn.reshape(ROWS, 512)
    emb2 = jnp.maximum(
        jnp.dot(h2, w3_ref[...], preferred_element_type=jnp.float32)
        + b3_ref[...], 0.0)
    out_ref[...] = (jnp.dot(emb2, w4_ref[...], preferred_element_type=jnp.float32)
                    + b4_ref[...] - sdn)


def _tc_mlp(sd, th, scale2, w1a, w1b, w1c, b1, w2t, b2, w3t, b3, w4t, b4,
            nw, nb):
    return pl.pallas_call(
        _mlp_body,
        out_shape=jax.ShapeDtypeStruct((ROWS, K), jnp.float32),
    )(sd, th, scale2, w1a, w1b, w1c, b1, w2t, b2, w3t, b3, w4t, b4, nw, nb)


def _scat_body(vals_ref, idx_ref, out_ref, rowbuf, ivb, vvb):
    pen = jnp.full((16,), PENALTY, jnp.float32)
    wid = lax.axis_index("c") * NS + lax.axis_index("s")

    def fill(i, _):
        rowbuf[pl.ds(i * 16, 16)] = pen
        return 0

    lax.fori_loop(0, NCHUNK, fill, 0)

    def do_row(r, _):
        row = wid * RPW + r
        pltpu.sync_copy(idx_ref.at[row], ivb)
        pltpu.sync_copy(vals_ref.at[row], vvb)
        for c in range(8):
            iv = ivb[pl.ds(c * 16, 16)]
            vv = vvb[pl.ds(c * 16, 16)]
            plsc.store_scatter(rowbuf, [iv], vv)
        pltpu.sync_copy(rowbuf, out_ref.at[row >> 4, row & 15])
        for c in range(8):
            iv = ivb[pl.ds(c * 16, 16)]
            plsc.store_scatter(rowbuf, [iv], pen)
        return 0

    lax.fori_loop(0, RPW, do_row, 0)


def _sc_scatter(vals, idx):
    kern = pl.kernel(
        _scat_body,
        out_type=jax.ShapeDtypeStruct((ROWS // 16, 16, N), jnp.float32),
        mesh=plsc.VectorSubcoreMesh(
            core_axis_name="c", subcore_axis_name="s",
            num_cores=NC, num_subcores=NS),
        compiler_params=pltpu.CompilerParams(needs_layout_passes=False),
        scratch_types=[
            pltpu.VMEM((N,), jnp.float32),
            pltpu.VMEM((K,), jnp.int32),
            pltpu.VMEM((K,), jnp.float32),
        ],
    )
    return kern(vals, idx)


@jax.jit
def kernel(dist, theta, scale, W1, b1, W2, b2, W3, b3, W4, b4, nw, nb):
    B, P, _ = dist.shape
    dist2 = dist.reshape(ROWS, N)
    theta_flat = theta.reshape(ROWS * N)
    scale2 = scale.reshape(ROWS, 1)

    sd, ix, th = _sc_topk(dist2, theta_flat)
    sd = sd.reshape(ROWS, K)
    ix = ix.reshape(ROWS, K)
    th = th.reshape(ROWS, K)

    w1a = W1[:, :K].T
    w1b = W1[:, K:2 * K].T
    w1c = W1[:, 2 * K].reshape(1, 256)
    out = _tc_mlp(sd, th, scale2, w1a, w1b, w1c, b1.reshape(1, 256),
                  W2.T, b2.reshape(1, 512), W3.T, b3.reshape(1, 256),
                  W4.T, b4.reshape(1, 128), nw, nb)

    return _sc_scatter(out, ix)
